# baseline (speedup 1.0000x reference)
import jax
import jax.numpy as jnp
from jax import lax
from jax.experimental import pallas as pl
from jax.experimental.pallas import tpu as pltpu
from jax.experimental.pallas import tpu_sc as plsc

NC, NS, L = 2, 16, 16
NW = NC * NS
B = 16384
D = 32
BPW = B // NW
CHUNK = 128
NCHUNK = BPW // CHUNK


def _sc_body(uidx_hbm, vidx_hbm, ut_hbm, vt_hbm, w_hbm, b_hbm, out_hbm,
             uidx_v, vidx_v, urows_v, mrows_v, wv_v, bv_v, out_v, sem):
    wid = lax.axis_index("s") * NC + lax.axis_index("c")
    base = wid * BPW

    pltpu.sync_copy(uidx_hbm.at[pl.ds(base, BPW)], uidx_v)
    pltpu.sync_copy(vidx_hbm.at[pl.ds(base, BPW)], vidx_v)
    pltpu.sync_copy(w_hbm, wv_v)
    pltpu.sync_copy(b_hbm, bv_v)

    urows2 = urows_v
    mrows2 = mrows_v
    copies = []
    for k in range(NCHUNK):
        sl = pl.ds(k * CHUNK, CHUNK)
        copies.append(pltpu.async_copy(ut_hbm.at[uidx_v.at[sl]], urows2.at[sl], sem))
        copies.append(pltpu.async_copy(vt_hbm.at[vidx_v.at[sl]], mrows2.at[sl], sem))
    for c in copies:
        c.wait()

    iota = lax.iota(jnp.int32, L)
    wv = wv_v[...]
    bv = bv_v[...]

    @pl.loop(0, BPW // L)
    def _group(g):
        rows = g * L + iota
        acc = jnp.zeros((L,), jnp.float32)
        for d in range(D):
            cols = jnp.full((L,), d, jnp.int32)
            acc = acc + (plsc.load_gather(urows_v, [rows, cols]) *
                         plsc.load_gather(mrows_v, [rows, cols]))
        z = acc * wv + bv
        sig = 1.0 / (1.0 + jnp.exp(-z))
        plsc.store_scatter(out_v, [rows], sig)

    pltpu.sync_copy(out_v, out_hbm.at[pl.ds(base, BPW)])


def kernel(x, user_table, video_table, fc_w, fc_b):
    uidx = x[0]
    vidx = x[1]
    wv = jnp.broadcast_to(fc_w.reshape(1), (L,)).astype(jnp.float32)
    bv = jnp.broadcast_to(fc_b.reshape(1), (L,)).astype(jnp.float32)
    mesh = plsc.VectorSubcoreMesh(core_axis_name="c", subcore_axis_name="s")
    f = pl.kernel(
        _sc_body,
        out_type=jax.ShapeDtypeStruct((B,), jnp.float32),
        mesh=mesh,
        scratch_types=[
            pltpu.VMEM((BPW,), jnp.int32),
            pltpu.VMEM((BPW,), jnp.int32),
            pltpu.VMEM((BPW, D), jnp.float32),
            pltpu.VMEM((BPW, D), jnp.float32),
            pltpu.VMEM((L,), jnp.float32),
            pltpu.VMEM((L,), jnp.float32),
            pltpu.VMEM((BPW,), jnp.float32),
            pltpu.SemaphoreType.DMA,
        ],
        compiler_params=pltpu.CompilerParams(
            needs_layout_passes=False, use_tc_tiling_on_sc=False),
    )
    out = f(uidx, vidx, user_table, video_table, wv, bv)
    return out.reshape(B, 1)


# speedup vs baseline: 3.6690x; 3.6690x over previous
import jax
import jax.numpy as jnp
from jax import lax
from jax.experimental import pallas as pl
from jax.experimental.pallas import tpu as pltpu
from jax.experimental.pallas import tpu_sc as plsc

NC, NS, L = 2, 16, 16
NW = NC * NS
B = 16384
D = 32
BPW = B // NW
CE = 4
NG = BPW // L
CPG = L // CE


def _sc_body(uidx_hbm, vidx_hbm, ut_hbm, vt_hbm, w_hbm, b_hbm, out_hbm,
             uidx_v, vidx_v, ubuf_v, mbuf_v, wv_v, bv_v,
             out_v, sems):
    wid = lax.axis_index("s") * NC + lax.axis_index("c")
    base = wid * BPW

    pltpu.sync_copy(uidx_hbm.at[pl.ds(base, BPW)], uidx_v.at[pl.ds(0, BPW)])
    pltpu.sync_copy(vidx_hbm.at[pl.ds(base, BPW)], vidx_v.at[pl.ds(0, BPW)])
    pltpu.sync_copy(w_hbm, wv_v)
    pltpu.sync_copy(b_hbm, bv_v)

    def fire(c, par):
        uvec = uidx_v[pl.ds(c * CE, L)] & -128
        vvec = vidx_v[pl.ds(c * CE, L)] & -128
        for e in range(CE):
            uj = pl.multiple_of(uvec[e], 128)
            vj = pl.multiple_of(vvec[e], 128)
            pltpu.async_copy(ut_hbm.at[:, pl.ds(uj, 128)],
                             ubuf_v.at[par, e], sems.at[par])
            pltpu.async_copy(vt_hbm.at[:, pl.ds(vj, 128)],
                             mbuf_v.at[par, e], sems.at[par])

    def drain(par):
        for e in range(CE):
            pltpu.make_async_copy(ut_hbm.at[:, pl.ds(0, 128)],
                                  ubuf_v.at[par, e], sems.at[par]).wait()
            pltpu.make_async_copy(vt_hbm.at[:, pl.ds(0, 128)],
                                  mbuf_v.at[par, e], sems.at[par]).wait()

    iota = lax.iota(jnp.int32, L)
    d_lo = lax.iota(jnp.int32, L)
    d_hi = d_lo + L
    wv = wv_v[...]
    bv = bv_v[...]

    fire(0, 0)

    @pl.loop(0, NG)
    def _group(g):
        acc = jnp.zeros((L,), jnp.float32)
        for q in range(CPG):
            c = g * CPG + q
            par = lax.rem(c, 2)

            @pl.when(c + 1 < NG * CPG)
            def _():
                fire(c + 1, 1 - par)

            drain(par)
            ulanes = uidx_v[pl.ds(c * CE, L)] & 127
            vlanes = vidx_v[pl.ds(c * CE, L)] & 127
            a = acc
            for e in range(CE):
                ku = ulanes[e] + jnp.zeros((L,), jnp.int32)
                kv = vlanes[e] + jnp.zeros((L,), jnp.int32)
                u0 = plsc.load_gather(ubuf_v.at[par, e], [d_lo, ku])
                u1 = plsc.load_gather(ubuf_v.at[par, e], [d_hi, ku])
                m0 = plsc.load_gather(mbuf_v.at[par, e], [d_lo, kv])
                m1 = plsc.load_gather(mbuf_v.at[par, e], [d_hi, kv])
                s = jnp.sum(u0 * m0 + u1 * m1)
                lane = q * CE + e
                a = a + jnp.where(iota == lane, s, 0.0)
            acc = a
        z = acc * wv + bv
        sig = 1.0 / (1.0 + jnp.exp(-z))
        plsc.store_scatter(out_v, [g * L + iota], sig)

    pltpu.sync_copy(out_v, out_hbm.at[pl.ds(base, BPW)])


def kernel(x, user_table, video_table, fc_w, fc_b):
    uidx = x[0]
    vidx = x[1]
    utt = user_table.T
    vtt = video_table.T
    wv = jnp.broadcast_to(fc_w.reshape(1), (L,)).astype(jnp.float32)
    bv = jnp.broadcast_to(fc_b.reshape(1), (L,)).astype(jnp.float32)
    mesh = plsc.VectorSubcoreMesh(core_axis_name="c", subcore_axis_name="s")
    f = pl.kernel(
        _sc_body,
        out_type=jax.ShapeDtypeStruct((B,), jnp.float32),
        mesh=mesh,
        scratch_types=[
            pltpu.VMEM((BPW + L,), jnp.int32),
            pltpu.VMEM((BPW + L,), jnp.int32),
            pltpu.VMEM((2, CE, D, 128), jnp.float32),
            pltpu.VMEM((2, CE, D, 128), jnp.float32),
            pltpu.VMEM((L,), jnp.float32),
            pltpu.VMEM((L,), jnp.float32),
            pltpu.VMEM((BPW,), jnp.float32),
            pltpu.SemaphoreType.DMA((2,)),
        ],
        compiler_params=pltpu.CompilerParams(
            needs_layout_passes=False, use_tc_tiling_on_sc=True),
    )
    out = f(uidx, vidx, utt, vtt, wv, bv)
    return out.reshape(B, 1)


# speedup vs baseline: 3.9332x; 1.0720x over previous
import jax
import jax.numpy as jnp
from jax import lax
from jax.experimental import pallas as pl
from jax.experimental.pallas import tpu as pltpu
from jax.experimental.pallas import tpu_sc as plsc

NC, NS, L = 2, 16, 16
NW = NC * NS
B = 16384
D = 32
BPW = B // NW
CE = 4
NG = BPW // L
CPG = L // CE
NBUF = 3


def _sc_body(uidx_hbm, vidx_hbm, ut_hbm, vt_hbm, w_hbm, b_hbm, out_hbm,
             uidx_v, vidx_v, ubuf_v, mbuf_v, wv_v, bv_v,
             out_v, sems):
    wid = lax.axis_index("s") * NC + lax.axis_index("c")
    base = wid * BPW

    pltpu.sync_copy(uidx_hbm.at[pl.ds(base, BPW)], uidx_v.at[pl.ds(0, BPW)])
    pltpu.sync_copy(vidx_hbm.at[pl.ds(base, BPW)], vidx_v.at[pl.ds(0, BPW)])
    pltpu.sync_copy(w_hbm, wv_v)
    pltpu.sync_copy(b_hbm, bv_v)

    def fire(c, par):
        uvec = uidx_v[pl.ds(c * CE, L)] & -128
        vvec = vidx_v[pl.ds(c * CE, L)] & -128
        for e in range(CE):
            uj = pl.multiple_of(uvec[e], 128)
            vj = pl.multiple_of(vvec[e], 128)
            pltpu.async_copy(ut_hbm.at[:, pl.ds(uj, 128)],
                             ubuf_v.at[par, e], sems.at[par])
            pltpu.async_copy(vt_hbm.at[:, pl.ds(vj, 128)],
                             mbuf_v.at[par, e], sems.at[par])

    def drain(par):
        for e in range(CE):
            pltpu.make_async_copy(ut_hbm.at[:, pl.ds(0, 128)],
                                  ubuf_v.at[par, e], sems.at[par]).wait()
            pltpu.make_async_copy(vt_hbm.at[:, pl.ds(0, 128)],
                                  mbuf_v.at[par, e], sems.at[par]).wait()

    iota = lax.iota(jnp.int32, L)
    d_lo = lax.iota(jnp.int32, L)
    d_hi = d_lo + L
    wv = wv_v[...]
    bv = bv_v[...]

    fire(0, 0)
    fire(1, 1)

    @pl.loop(0, NG)
    def _group(g):
        acc = jnp.zeros((L,), jnp.float32)
        for q in range(CPG):
            c = g * CPG + q
            par = lax.rem(c, NBUF)

            @pl.when(c + 2 < NG * CPG)
            def _():
                fire(c + 2, lax.rem(c + 2, NBUF))

            drain(par)
            ulanes = uidx_v[pl.ds(c * CE, L)] & 127
            vlanes = vidx_v[pl.ds(c * CE, L)] & 127
            a = acc
            for e in range(CE):
                ku = ulanes[e] + jnp.zeros((L,), jnp.int32)
                kv = vlanes[e] + jnp.zeros((L,), jnp.int32)
                u0 = plsc.load_gather(ubuf_v.at[par, e], [d_lo, ku])
                u1 = plsc.load_gather(ubuf_v.at[par, e], [d_hi, ku])
                m0 = plsc.load_gather(mbuf_v.at[par, e], [d_lo, kv])
                m1 = plsc.load_gather(mbuf_v.at[par, e], [d_hi, kv])
                s = jnp.sum(u0 * m0 + u1 * m1)
                lane = q * CE + e
                a = a + jnp.where(iota == lane, s, 0.0)
            acc = a
        z = acc * wv + bv
        sig = 1.0 / (1.0 + jnp.exp(-z))
        plsc.store_scatter(out_v, [g * L + iota], sig)

    pltpu.sync_copy(out_v, out_hbm.at[pl.ds(base, BPW)])


def kernel(x, user_table, video_table, fc_w, fc_b):
    uidx = x[0]
    vidx = x[1]
    utt = user_table.T
    vtt = video_table.T
    wv = jnp.broadcast_to(fc_w.reshape(1), (L,)).astype(jnp.float32)
    bv = jnp.broadcast_to(fc_b.reshape(1), (L,)).astype(jnp.float32)
    mesh = plsc.VectorSubcoreMesh(core_axis_name="c", subcore_axis_name="s")
    f = pl.kernel(
        _sc_body,
        out_type=jax.ShapeDtypeStruct((B,), jnp.float32),
        mesh=mesh,
        scratch_types=[
            pltpu.VMEM((BPW + L,), jnp.int32),
            pltpu.VMEM((BPW + L,), jnp.int32),
            pltpu.VMEM((NBUF, CE, D, 128), jnp.float32),
            pltpu.VMEM((NBUF, CE, D, 128), jnp.float32),
            pltpu.VMEM((L,), jnp.float32),
            pltpu.VMEM((L,), jnp.float32),
            pltpu.VMEM((BPW,), jnp.float32),
            pltpu.SemaphoreType.DMA((NBUF,)),
        ],
        compiler_params=pltpu.CompilerParams(
            needs_layout_passes=False, use_tc_tiling_on_sc=True),
    )
    out = f(uidx, vidx, utt, vtt, wv, bv)
    return out.reshape(B, 1)


# speedup vs baseline: 3.9445x; 1.0029x over previous
import jax
import jax.numpy as jnp
from jax import lax
from jax.experimental import pallas as pl
from jax.experimental.pallas import tpu as pltpu
from jax.experimental.pallas import tpu_sc as plsc

NC, NS, L = 2, 16, 16
NW = NC * NS
B = 16384
D = 32
BPW = B // NW
CE = 4
NG = BPW // L
CPG = L // CE
NBUF = 3


def _sc_body(uidx_hbm, vidx_hbm, ut_hbm, vt_hbm, w_hbm, b_hbm, out_hbm,
             uidx_v, vidx_v, ubuf_v, mbuf_v, wv_v, bv_v,
             out_v, sems):
    wid = lax.axis_index("s") * NC + lax.axis_index("c")
    base = wid * BPW

    pltpu.sync_copy(uidx_hbm.at[pl.ds(base, BPW)], uidx_v.at[pl.ds(0, BPW)])
    pltpu.sync_copy(vidx_hbm.at[pl.ds(base, BPW)], vidx_v.at[pl.ds(0, BPW)])
    pltpu.sync_copy(w_hbm, wv_v)
    pltpu.sync_copy(b_hbm, bv_v)

    def fire(c, par):
        uvec = uidx_v[pl.ds(c * CE, L)] & -128
        vvec = vidx_v[pl.ds(c * CE, L)] & -128
        for e in range(CE):
            uj = pl.multiple_of(uvec[e], 128)
            vj = pl.multiple_of(vvec[e], 128)
            for sb in range(D // 8):
                pltpu.async_copy(
                    ut_hbm.at[pl.ds(sb * 8, 8), pl.ds(uj, 128)],
                    ubuf_v.at[par, e, pl.ds(sb * 8, 8)], sems.at[par])
                pltpu.async_copy(
                    vt_hbm.at[pl.ds(sb * 8, 8), pl.ds(vj, 128)],
                    mbuf_v.at[par, e, pl.ds(sb * 8, 8)], sems.at[par])

    def drain(par):
        for e in range(CE):
            pltpu.make_async_copy(ut_hbm.at[:, pl.ds(0, 128)],
                                  ubuf_v.at[par, e], sems.at[par]).wait()
            pltpu.make_async_copy(vt_hbm.at[:, pl.ds(0, 128)],
                                  mbuf_v.at[par, e], sems.at[par]).wait()

    iota = lax.iota(jnp.int32, L)
    d_lo = lax.iota(jnp.int32, L)
    d_hi = d_lo + L
    wv = wv_v[...]
    bv = bv_v[...]

    fire(0, 0)
    fire(1, 1)

    @pl.loop(0, NG)
    def _group(g):
        acc = jnp.zeros((L,), jnp.float32)
        for q in range(CPG):
            c = g * CPG + q
            par = lax.rem(c, NBUF)

            @pl.when(c + 2 < NG * CPG)
            def _():
                fire(c + 2, lax.rem(c + 2, NBUF))

            drain(par)
            ulanes = uidx_v[pl.ds(c * CE, L)] & 127
            vlanes = vidx_v[pl.ds(c * CE, L)] & 127
            a = acc
            for e in range(CE):
                ku = ulanes[e] + jnp.zeros((L,), jnp.int32)
                kv = vlanes[e] + jnp.zeros((L,), jnp.int32)
                u0 = plsc.load_gather(ubuf_v.at[par, e], [d_lo, ku])
                u1 = plsc.load_gather(ubuf_v.at[par, e], [d_hi, ku])
                m0 = plsc.load_gather(mbuf_v.at[par, e], [d_lo, kv])
                m1 = plsc.load_gather(mbuf_v.at[par, e], [d_hi, kv])
                s = jnp.sum(u0 * m0 + u1 * m1)
                lane = q * CE + e
                a = a + jnp.where(iota == lane, s, 0.0)
            acc = a
        z = acc * wv + bv
        sig = 1.0 / (1.0 + jnp.exp(-z))
        plsc.store_scatter(out_v, [g * L + iota], sig)

    pltpu.sync_copy(out_v, out_hbm.at[pl.ds(base, BPW)])


def kernel(x, user_table, video_table, fc_w, fc_b):
    uidx = x[0]
    vidx = x[1]
    utt = user_table.T
    vtt = video_table.T
    wv = jnp.broadcast_to(fc_w.reshape(1), (L,)).astype(jnp.float32)
    bv = jnp.broadcast_to(fc_b.reshape(1), (L,)).astype(jnp.float32)
    mesh = plsc.VectorSubcoreMesh(core_axis_name="c", subcore_axis_name="s")
    f = pl.kernel(
        _sc_body,
        out_type=jax.ShapeDtypeStruct((B,), jnp.float32),
        mesh=mesh,
        scratch_types=[
            pltpu.VMEM((BPW + L,), jnp.int32),
            pltpu.VMEM((BPW + L,), jnp.int32),
            pltpu.VMEM((NBUF, CE, D, 128), jnp.float32),
            pltpu.VMEM((NBUF, CE, D, 128), jnp.float32),
            pltpu.VMEM((L,), jnp.float32),
            pltpu.VMEM((L,), jnp.float32),
            pltpu.VMEM((BPW,), jnp.float32),
            pltpu.SemaphoreType.DMA((NBUF,)),
        ],
        compiler_params=pltpu.CompilerParams(
            needs_layout_passes=False, use_tc_tiling_on_sc=True),
    )
    out = f(uidx, vidx, utt, vtt, wv, bv)
    return out.reshape(B, 1)
